# baseline (speedup 1.0000x reference)
import functools

import numpy as np
import jax
import jax.numpy as jnp
from jax import lax
from jax.experimental import pallas as pl
from jax.experimental.pallas import tpu as pltpu
from jax.experimental.pallas import tpu_sc as plsc

_N = 10000
_E = 320000
_D = 128

_NC = 2
_NS = 16
_NW = _NC * _NS
_CH = 128
_NCHUNK = 80
_NPAIR = _NCHUNK // 2
_EPW = _NCHUNK * _CH
_E2 = _NW * _EPW
_GARB = _N
_NACC = _N + 8
_RPT = 632
_SHIFT = 14


def _sc_segment_sum(h, packed3, zer):
    mesh = plsc.VectorSubcoreMesh(core_axis_name="c", subcore_axis_name="s")

    def body(h_hbm, packed_hbm, zer_hbm, out_hbm,
             packed_v, src0, dst0, src1, dst1, rows0, rows1, acc_sh,
             sem_g0, sem_g1, sem_s0, sem_s1):
        c = lax.axis_index("c")
        s = lax.axis_index("s")
        wid = s * _NC + c

        def on_my_rows(fn, last_n):
            @pl.when(s < _NS - 1)
            def _():
                fn(pl.ds(pl.multiple_of(s * _RPT, 8), _RPT))

            @pl.when(s == _NS - 1)
            def _():
                fn(pl.ds((_NS - 1) * _RPT, last_n))

        on_my_rows(lambda rsl: pltpu.sync_copy(zer_hbm.at[rsl], acc_sh.at[rsl]),
                   _NACC - 15 * _RPT)
        plsc.subcore_barrier()
        pltpu.sync_copy(packed_hbm.at[wid], packed_v)

        def unpack(j, sbuf, dbuf):
            for k in range(_CH // 16):
                p = packed_v[j, pl.ds(k * 16, 16)]
                sbuf[0, pl.ds(k * 16, 16)] = p & (2**_SHIFT - 1)
                dbuf[0, pl.ds(k * 16, 16)] = lax.shift_right_logical(p, _SHIFT)

        unpack(0, src0, dst0)
        pltpu.async_copy(h_hbm.at[src0.at[0]], rows0, sem_g0)

        def step(t, carry):
            a = 2 * t
            b = a + 1

            @pl.when(t > 0)
            def _():
                pltpu.make_async_copy(rows1, acc_sh.at[dst1.at[0]], sem_s1).wait()

            unpack(b, src1, dst1)
            pltpu.async_copy(h_hbm.at[src1.at[0]], rows1, sem_g1)
            pltpu.make_async_copy(h_hbm.at[src0.at[0]], rows0, sem_g0).wait()
            pltpu.async_copy(rows0, acc_sh.at[dst0.at[0]], sem_s0, add=True)
            pltpu.make_async_copy(rows0, acc_sh.at[dst0.at[0]], sem_s0).wait()

            @pl.when(t < _NPAIR - 1)
            def _():
                unpack(a + 2, src0, dst0)
                pltpu.async_copy(h_hbm.at[src0.at[0]], rows0, sem_g0)

            pltpu.make_async_copy(h_hbm.at[src1.at[0]], rows1, sem_g1).wait()
            pltpu.async_copy(rows1, acc_sh.at[dst1.at[0]], sem_s1, add=True)
            return carry

        lax.fori_loop(0, _NPAIR, step, 0)
        pltpu.make_async_copy(rows1, acc_sh.at[dst1.at[0]], sem_s1).wait()
        plsc.subcore_barrier()
        on_my_rows(lambda rsl: pltpu.sync_copy(acc_sh.at[rsl], out_hbm.at[c, rsl]),
                   _N - 15 * _RPT)

    f = pl.kernel(
        body,
        out_type=jax.ShapeDtypeStruct((_NC, _N, _D), jnp.float32),
        mesh=mesh,
        scratch_types=[
            pltpu.VMEM((_NCHUNK, _CH), jnp.int32),
            pltpu.VMEM((1, _CH), jnp.int32),
            pltpu.VMEM((1, _CH), jnp.int32),
            pltpu.VMEM((1, _CH), jnp.int32),
            pltpu.VMEM((1, _CH), jnp.int32),
            pltpu.VMEM((_CH, _D), jnp.float32),
            pltpu.VMEM((_CH, _D), jnp.float32),
            pltpu.VMEM_SHARED((_NACC, _D), jnp.float32),
            pltpu.SemaphoreType.DMA,
            pltpu.SemaphoreType.DMA,
            pltpu.SemaphoreType.DMA,
            pltpu.SemaphoreType.DMA,
        ],
    )
    return f(h, packed3, zer)


_BR = 1000


def _tc_mlp_bn(scale, h, agg2, Wa, ba, Wb, bb, gamma, beta):
    def body(sc_ref, h_ref, a0_ref, a1_ref, wa_ref, ba_ref, wb_ref, bb_ref,
             g_ref, be_ref, o_ref):
        z = sc_ref[0] * h_ref[...] + a0_ref[...] + a1_ref[...]
        z = jnp.dot(z, wa_ref[...], preferred_element_type=jnp.float32)
        z = jnp.maximum(z + ba_ref[...], 0.0)
        y = jnp.dot(z, wb_ref[...], preferred_element_type=jnp.float32)
        y = y + bb_ref[...]
        y = y * (g_ref[...] * np.float32(1.0 / np.sqrt(1.0 + 1e-5))) + be_ref[...]
        o_ref[...] = jnp.where(y >= 0.0, y, 0.01 * y)

    dout = Wb.shape[1]
    return pl.pallas_call(
        body,
        grid=(_N // _BR,),
        in_specs=[
            pl.BlockSpec(memory_space=pltpu.SMEM),
            pl.BlockSpec((_BR, _D), lambda i: (i, 0)),
            pl.BlockSpec((_BR, _D), lambda i: (i, 0)),
            pl.BlockSpec((_BR, _D), lambda i: (i, 0)),
            pl.BlockSpec((_D, _D), lambda i: (0, 0)),
            pl.BlockSpec((1, _D), lambda i: (0, 0)),
            pl.BlockSpec((_D, dout), lambda i: (0, 0)),
            pl.BlockSpec((1, dout), lambda i: (0, 0)),
            pl.BlockSpec((1, dout), lambda i: (0, 0)),
            pl.BlockSpec((1, dout), lambda i: (0, 0)),
        ],
        out_specs=pl.BlockSpec((_BR, dout), lambda i: (i, 0)),
        out_shape=jax.ShapeDtypeStruct((_N, dout), jnp.float32),
    )(scale, h, agg2[0], agg2[1], Wa, ba.reshape(1, -1), Wb,
      bb.reshape(1, -1), gamma.reshape(1, -1), beta.reshape(1, -1))


def _tc_mlp(scale, h, agg2, Wa, ba, Wb, bb):
    def body(sc_ref, h_ref, a0_ref, a1_ref, wa_ref, ba_ref, wb_ref, bb_ref,
             o_ref):
        z = sc_ref[0] * h_ref[...] + a0_ref[...] + a1_ref[...]
        z = jnp.dot(z, wa_ref[...], preferred_element_type=jnp.float32)
        z = jnp.maximum(z + ba_ref[...], 0.0)
        y = jnp.dot(z, wb_ref[...], preferred_element_type=jnp.float32)
        o_ref[...] = y + bb_ref[...]

    dout = Wb.shape[1]
    return pl.pallas_call(
        body,
        grid=(_N // _BR,),
        in_specs=[
            pl.BlockSpec(memory_space=pltpu.SMEM),
            pl.BlockSpec((_BR, _D), lambda i: (i, 0)),
            pl.BlockSpec((_BR, _D), lambda i: (i, 0)),
            pl.BlockSpec((_BR, _D), lambda i: (i, 0)),
            pl.BlockSpec((_D, _D), lambda i: (0, 0)),
            pl.BlockSpec((1, _D), lambda i: (0, 0)),
            pl.BlockSpec((_D, dout), lambda i: (0, 0)),
            pl.BlockSpec((1, dout), lambda i: (0, 0)),
        ],
        out_specs=pl.BlockSpec((_BR, dout), lambda i: (i, 0)),
        out_shape=jax.ShapeDtypeStruct((_N, dout), jnp.float32),
    )(scale, h, agg2[0], agg2[1], Wa, ba.reshape(1, -1), Wb,
      bb.reshape(1, -1))


def kernel(x, edge_index, W0a, b0a, W0b, b0b, eps0, W1a, b1a, W1b, b1b,
           W2a, b2a, W2b, b2b, eps2, gamma, beta):
    packed = edge_index[0] + edge_index[1] * (2**_SHIFT)
    pad = jnp.full((_E2 - _E,), _GARB * (2**_SHIFT), jnp.int32)
    packed3 = jnp.concatenate([packed, pad]).reshape(_NW, _NCHUNK, _CH)
    zer = jnp.zeros((_NACC, _D), jnp.float32)

    s0 = (1.0 + eps0).reshape(1)
    s1 = jnp.ones((1,), jnp.float32)
    s2 = (1.0 + eps2).reshape(1)

    agg = _sc_segment_sum(x, packed3, zer)
    h = _tc_mlp_bn(s0, x, agg, W0a, b0a, W0b, b0b, gamma, beta)
    agg = _sc_segment_sum(h, packed3, zer)
    h = _tc_mlp_bn(s1, h, agg, W1a, b1a, W1b, b1b, gamma, beta)
    agg = _sc_segment_sum(h, packed3, zer)
    return _tc_mlp(s2, h, agg, W2a, b2a, W2b, b2b)


# speedup vs baseline: 1.0286x; 1.0286x over previous
import functools

import numpy as np
import jax
import jax.numpy as jnp
from jax import lax
from jax.experimental import pallas as pl
from jax.experimental.pallas import tpu as pltpu
from jax.experimental.pallas import tpu_sc as plsc

_N = 10000
_E = 320000
_D = 128

_NC = 2
_NS = 16
_NW = _NC * _NS
_CH = 128
_NCHUNK = 80
_NPAIR = _NCHUNK // 2
_EPW = _NCHUNK * _CH
_E2 = _NW * _EPW
_NPADROW = 128
_NACC = _N + _NPADROW
_RPT = 632
_SHIFT = 14


def _sc_segment_sum(h, packed3, zer):
    mesh = plsc.VectorSubcoreMesh(core_axis_name="c", subcore_axis_name="s")

    def body(h_hbm, packed_hbm, zer_hbm, out_hbm,
             packed_v, src0, dst0, src1, dst1, rows0, rows1, acc_sh,
             sem_g0, sem_g1, sem_s0, sem_s1):
        c = lax.axis_index("c")
        s = lax.axis_index("s")
        wid = s * _NC + c

        def on_my_rows(fn, last_n):
            @pl.when(s < _NS - 1)
            def _():
                fn(pl.ds(pl.multiple_of(s * _RPT, 8), _RPT))

            @pl.when(s == _NS - 1)
            def _():
                fn(pl.ds((_NS - 1) * _RPT, last_n))

        on_my_rows(lambda rsl: pltpu.sync_copy(zer_hbm.at[rsl], acc_sh.at[rsl]),
                   _NACC - 15 * _RPT)
        plsc.subcore_barrier()
        pltpu.sync_copy(packed_hbm.at[wid], packed_v)

        def unpack(j, sbuf, dbuf):
            for k in range(_CH // 16):
                p = packed_v[j, pl.ds(k * 16, 16)]
                sbuf[0, pl.ds(k * 16, 16)] = p & (2**_SHIFT - 1)
                dbuf[0, pl.ds(k * 16, 16)] = lax.shift_right_logical(p, _SHIFT)

        unpack(0, src0, dst0)
        pltpu.async_copy(h_hbm.at[src0.at[0]], rows0, sem_g0)

        def step(t, carry):
            a = 2 * t
            b = a + 1

            @pl.when(t > 0)
            def _():
                pltpu.make_async_copy(rows1, acc_sh.at[dst1.at[0]], sem_s1).wait()

            unpack(b, src1, dst1)
            pltpu.async_copy(h_hbm.at[src1.at[0]], rows1, sem_g1)
            pltpu.make_async_copy(h_hbm.at[src0.at[0]], rows0, sem_g0).wait()
            pltpu.async_copy(rows0, acc_sh.at[dst0.at[0]], sem_s0, add=True)
            pltpu.make_async_copy(rows0, acc_sh.at[dst0.at[0]], sem_s0).wait()

            @pl.when(t < _NPAIR - 1)
            def _():
                unpack(a + 2, src0, dst0)
                pltpu.async_copy(h_hbm.at[src0.at[0]], rows0, sem_g0)

            pltpu.make_async_copy(h_hbm.at[src1.at[0]], rows1, sem_g1).wait()
            pltpu.async_copy(rows1, acc_sh.at[dst1.at[0]], sem_s1, add=True)
            return carry

        lax.fori_loop(0, _NPAIR, step, 0)
        pltpu.make_async_copy(rows1, acc_sh.at[dst1.at[0]], sem_s1).wait()
        plsc.subcore_barrier()
        on_my_rows(lambda rsl: pltpu.sync_copy(acc_sh.at[rsl], out_hbm.at[c, rsl]),
                   _N - 15 * _RPT)

    f = pl.kernel(
        body,
        out_type=jax.ShapeDtypeStruct((_NC, _N, _D), jnp.float32),
        mesh=mesh,
        scratch_types=[
            pltpu.VMEM((_NCHUNK, _CH), jnp.int32),
            pltpu.VMEM((1, _CH), jnp.int32),
            pltpu.VMEM((1, _CH), jnp.int32),
            pltpu.VMEM((1, _CH), jnp.int32),
            pltpu.VMEM((1, _CH), jnp.int32),
            pltpu.VMEM((_CH, _D), jnp.float32),
            pltpu.VMEM((_CH, _D), jnp.float32),
            pltpu.VMEM_SHARED((_NACC, _D), jnp.float32),
            pltpu.SemaphoreType.DMA,
            pltpu.SemaphoreType.DMA,
            pltpu.SemaphoreType.DMA,
            pltpu.SemaphoreType.DMA,
        ],
    )
    return f(h, packed3, zer)


_BR = 1000


def _tc_mlp_bn(scale, h, agg2, Wa, ba, Wb, bb, gamma, beta):
    def body(sc_ref, h_ref, a0_ref, a1_ref, wa_ref, ba_ref, wb_ref, bb_ref,
             g_ref, be_ref, o_ref):
        z = sc_ref[0] * h_ref[...] + a0_ref[...] + a1_ref[...]
        z = jnp.dot(z, wa_ref[...], preferred_element_type=jnp.float32)
        z = jnp.maximum(z + ba_ref[...], 0.0)
        y = jnp.dot(z, wb_ref[...], preferred_element_type=jnp.float32)
        y = y + bb_ref[...]
        y = y * (g_ref[...] * np.float32(1.0 / np.sqrt(1.0 + 1e-5))) + be_ref[...]
        o_ref[...] = jnp.where(y >= 0.0, y, 0.01 * y)

    dout = Wb.shape[1]
    return pl.pallas_call(
        body,
        grid=(_N // _BR,),
        in_specs=[
            pl.BlockSpec(memory_space=pltpu.SMEM),
            pl.BlockSpec((_BR, _D), lambda i: (i, 0)),
            pl.BlockSpec((_BR, _D), lambda i: (i, 0)),
            pl.BlockSpec((_BR, _D), lambda i: (i, 0)),
            pl.BlockSpec((_D, _D), lambda i: (0, 0)),
            pl.BlockSpec((1, _D), lambda i: (0, 0)),
            pl.BlockSpec((_D, dout), lambda i: (0, 0)),
            pl.BlockSpec((1, dout), lambda i: (0, 0)),
            pl.BlockSpec((1, dout), lambda i: (0, 0)),
            pl.BlockSpec((1, dout), lambda i: (0, 0)),
        ],
        out_specs=pl.BlockSpec((_BR, dout), lambda i: (i, 0)),
        out_shape=jax.ShapeDtypeStruct((_N, dout), jnp.float32),
    )(scale, h, agg2[0], agg2[1], Wa, ba.reshape(1, -1), Wb,
      bb.reshape(1, -1), gamma.reshape(1, -1), beta.reshape(1, -1))


def _tc_mlp(scale, h, agg2, Wa, ba, Wb, bb):
    def body(sc_ref, h_ref, a0_ref, a1_ref, wa_ref, ba_ref, wb_ref, bb_ref,
             o_ref):
        z = sc_ref[0] * h_ref[...] + a0_ref[...] + a1_ref[...]
        z = jnp.dot(z, wa_ref[...], preferred_element_type=jnp.float32)
        z = jnp.maximum(z + ba_ref[...], 0.0)
        y = jnp.dot(z, wb_ref[...], preferred_element_type=jnp.float32)
        o_ref[...] = y + bb_ref[...]

    dout = Wb.shape[1]
    return pl.pallas_call(
        body,
        grid=(_N // _BR,),
        in_specs=[
            pl.BlockSpec(memory_space=pltpu.SMEM),
            pl.BlockSpec((_BR, _D), lambda i: (i, 0)),
            pl.BlockSpec((_BR, _D), lambda i: (i, 0)),
            pl.BlockSpec((_BR, _D), lambda i: (i, 0)),
            pl.BlockSpec((_D, _D), lambda i: (0, 0)),
            pl.BlockSpec((1, _D), lambda i: (0, 0)),
            pl.BlockSpec((_D, dout), lambda i: (0, 0)),
            pl.BlockSpec((1, dout), lambda i: (0, 0)),
        ],
        out_specs=pl.BlockSpec((_BR, dout), lambda i: (i, 0)),
        out_shape=jax.ShapeDtypeStruct((_N, dout), jnp.float32),
    )(scale, h, agg2[0], agg2[1], Wa, ba.reshape(1, -1), Wb,
      bb.reshape(1, -1))


def kernel(x, edge_index, W0a, b0a, W0b, b0b, eps0, W1a, b1a, W1b, b1b,
           W2a, b2a, W2b, b2b, eps2, gamma, beta):
    packed = edge_index[0] + edge_index[1] * (2**_SHIFT)
    pad = jnp.asarray((_N + np.arange(_E2 - _E) % _NPADROW) * (2**_SHIFT),
                      jnp.int32)
    packed3 = jnp.concatenate([packed, pad]).reshape(_NW, _NCHUNK, _CH)
    zer = jnp.zeros((_NACC, _D), jnp.float32)

    s0 = (1.0 + eps0).reshape(1)
    s1 = jnp.ones((1,), jnp.float32)
    s2 = (1.0 + eps2).reshape(1)

    agg = _sc_segment_sum(x, packed3, zer)
    h = _tc_mlp_bn(s0, x, agg, W0a, b0a, W0b, b0b, gamma, beta)
    agg = _sc_segment_sum(h, packed3, zer)
    h = _tc_mlp_bn(s1, h, agg, W1a, b1a, W1b, b1b, gamma, beta)
    agg = _sc_segment_sum(h, packed3, zer)
    return _tc_mlp(s2, h, agg, W2a, b2a, W2b, b2b)


# speedup vs baseline: 3.5391x; 3.4407x over previous
import functools

import numpy as np
import jax
import jax.numpy as jnp
from jax import lax
from jax.experimental import pallas as pl
from jax.experimental.pallas import tpu as pltpu
from jax.experimental.pallas import tpu_sc as plsc

_N = 10000
_E = 320000
_D = 128

_NC = 2
_NS = 16
_NW = _NC * _NS
_CH = 128
_NCHUNK = 80
_NPAIR = _NCHUNK // 2
_EPW = _NCHUNK * _CH
_E2 = _NW * _EPW
_NPADROW = 128
_NACC = _N + _NPADROW
_RPT = 632
_SHIFT = 14


def _sc_segment_sum(h, packed3, zer):
    mesh = plsc.VectorSubcoreMesh(core_axis_name="c", subcore_axis_name="s")

    def body(h_hbm, packed_hbm, zer_hbm, out_hbm,
             packed_v, src0, dst0, src1, dst1, rows0, rows1, acc_sh,
             sem_g0, sem_g1, sem_s0, sem_s1):
        c = lax.axis_index("c")
        s = lax.axis_index("s")
        wid = s * _NC + c

        def on_my_rows(fn, last_n):
            @pl.when(s < _NS - 1)
            def _():
                fn(pl.ds(pl.multiple_of(s * _RPT, 8), _RPT))

            @pl.when(s == _NS - 1)
            def _():
                fn(pl.ds((_NS - 1) * _RPT, last_n))

        on_my_rows(lambda rsl: pltpu.sync_copy(zer_hbm.at[rsl], acc_sh.at[rsl]),
                   _NACC - 15 * _RPT)
        plsc.subcore_barrier()
        pltpu.sync_copy(packed_hbm.at[wid], packed_v)

        def unpack(j, sbuf, dbuf):
            for k in range(_CH // 16):
                p = packed_v[j, pl.ds(k * 16, 16)]
                sbuf[0, pl.ds(k * 16, 16)] = p & (2**_SHIFT - 1)
                dbuf[0, pl.ds(k * 16, 16)] = lax.shift_right_logical(p, _SHIFT)

        unpack(0, src0, dst0)
        pltpu.async_copy(h_hbm.at[src0.at[0]], rows0, sem_g0)

        def step(t, carry):
            a = 2 * t
            b = a + 1

            @pl.when(t > 0)
            def _():
                pltpu.make_async_copy(rows1, acc_sh.at[dst1.at[0]], sem_s1).wait()

            unpack(b, src1, dst1)
            pltpu.async_copy(h_hbm.at[src1.at[0]], rows1, sem_g1)
            pltpu.make_async_copy(h_hbm.at[src0.at[0]], rows0, sem_g0).wait()
            pltpu.async_copy(rows0, acc_sh.at[dst0.at[0]], sem_s0, add=True)
            pltpu.make_async_copy(rows0, acc_sh.at[dst0.at[0]], sem_s0).wait()

            @pl.when(t < _NPAIR - 1)
            def _():
                unpack(a + 2, src0, dst0)
                pltpu.async_copy(h_hbm.at[src0.at[0]], rows0, sem_g0)

            pltpu.make_async_copy(h_hbm.at[src1.at[0]], rows1, sem_g1).wait()
            pltpu.async_copy(rows1, acc_sh.at[dst1.at[0]], sem_s1, add=True)
            return carry

        lax.fori_loop(0, _NPAIR, step, 0)
        pltpu.make_async_copy(rows1, acc_sh.at[dst1.at[0]], sem_s1).wait()
        plsc.subcore_barrier()
        on_my_rows(lambda rsl: pltpu.sync_copy(acc_sh.at[rsl], out_hbm.at[c, rsl]),
                   _N - 15 * _RPT)

    f = pl.kernel(
        body,
        out_type=jax.ShapeDtypeStruct((_NC, _N, _D), jnp.float32),
        mesh=mesh,
        scratch_types=[
            pltpu.VMEM((_NCHUNK, _CH), jnp.int32),
            pltpu.VMEM((1, _CH), jnp.int32),
            pltpu.VMEM((1, _CH), jnp.int32),
            pltpu.VMEM((1, _CH), jnp.int32),
            pltpu.VMEM((1, _CH), jnp.int32),
            pltpu.VMEM((_CH, _D), jnp.float32),
            pltpu.VMEM((_CH, _D), jnp.float32),
            pltpu.VMEM_SHARED((_NACC, _D), jnp.float32),
            pltpu.SemaphoreType.DMA,
            pltpu.SemaphoreType.DMA,
            pltpu.SemaphoreType.DMA,
            pltpu.SemaphoreType.DMA,
        ],
    )
    return f(h, packed3, zer)


_BR = 1000


def _tc_mlp_bn(scale, h, agg2, Wa, ba, Wb, bb, gamma, beta):
    def body(sc_ref, h_ref, a0_ref, a1_ref, wa_ref, ba_ref, wb_ref, bb_ref,
             g_ref, be_ref, o_ref):
        z = sc_ref[0] * h_ref[...] + a0_ref[...] + a1_ref[...]
        z = jnp.dot(z, wa_ref[...], preferred_element_type=jnp.float32)
        z = jnp.maximum(z + ba_ref[...], 0.0)
        y = jnp.dot(z, wb_ref[...], preferred_element_type=jnp.float32)
        y = y + bb_ref[...]
        y = y * (g_ref[...] * np.float32(1.0 / np.sqrt(1.0 + 1e-5))) + be_ref[...]
        o_ref[...] = jnp.where(y >= 0.0, y, 0.01 * y)

    dout = Wb.shape[1]
    return pl.pallas_call(
        body,
        grid=(_N // _BR,),
        in_specs=[
            pl.BlockSpec(memory_space=pltpu.SMEM),
            pl.BlockSpec((_BR, _D), lambda i: (i, 0)),
            pl.BlockSpec((_BR, _D), lambda i: (i, 0)),
            pl.BlockSpec((_BR, _D), lambda i: (i, 0)),
            pl.BlockSpec((_D, _D), lambda i: (0, 0)),
            pl.BlockSpec((1, _D), lambda i: (0, 0)),
            pl.BlockSpec((_D, dout), lambda i: (0, 0)),
            pl.BlockSpec((1, dout), lambda i: (0, 0)),
            pl.BlockSpec((1, dout), lambda i: (0, 0)),
            pl.BlockSpec((1, dout), lambda i: (0, 0)),
        ],
        out_specs=pl.BlockSpec((_BR, dout), lambda i: (i, 0)),
        out_shape=jax.ShapeDtypeStruct((_N, dout), jnp.float32),
    )(scale, h, agg2[0], agg2[1], Wa, ba.reshape(1, -1), Wb,
      bb.reshape(1, -1), gamma.reshape(1, -1), beta.reshape(1, -1))


def _tc_mlp(scale, h, agg2, Wa, ba, Wb, bb):
    def body(sc_ref, h_ref, a0_ref, a1_ref, wa_ref, ba_ref, wb_ref, bb_ref,
             o_ref):
        z = sc_ref[0] * h_ref[...] + a0_ref[...] + a1_ref[...]
        z = jnp.dot(z, wa_ref[...], preferred_element_type=jnp.float32)
        z = jnp.maximum(z + ba_ref[...], 0.0)
        y = jnp.dot(z, wb_ref[...], preferred_element_type=jnp.float32)
        o_ref[...] = y + bb_ref[...]

    dout = Wb.shape[1]
    return pl.pallas_call(
        body,
        grid=(_N // _BR,),
        in_specs=[
            pl.BlockSpec(memory_space=pltpu.SMEM),
            pl.BlockSpec((_BR, _D), lambda i: (i, 0)),
            pl.BlockSpec((_BR, _D), lambda i: (i, 0)),
            pl.BlockSpec((_BR, _D), lambda i: (i, 0)),
            pl.BlockSpec((_D, _D), lambda i: (0, 0)),
            pl.BlockSpec((1, _D), lambda i: (0, 0)),
            pl.BlockSpec((_D, dout), lambda i: (0, 0)),
            pl.BlockSpec((1, dout), lambda i: (0, 0)),
        ],
        out_specs=pl.BlockSpec((_BR, dout), lambda i: (i, 0)),
        out_shape=jax.ShapeDtypeStruct((_N, dout), jnp.float32),
    )(scale, h, agg2[0], agg2[1], Wa, ba.reshape(1, -1), Wb,
      bb.reshape(1, -1))


def kernel(x, edge_index, W0a, b0a, W0b, b0b, eps0, W1a, b1a, W1b, b1b,
           W2a, b2a, W2b, b2b, eps2, gamma, beta):
    packed = edge_index[0] + edge_index[1] * (2**_SHIFT)
    _pi = np.arange(_E2 - _E)
    pad = jnp.asarray(_pi * 13 % _N + (_N + _pi % _NPADROW) * (2**_SHIFT),
                      jnp.int32)
    packed3 = jnp.concatenate([packed, pad]).reshape(_NW, _NCHUNK, _CH)
    zer = jnp.zeros((_NACC, _D), jnp.float32)

    s0 = (1.0 + eps0).reshape(1)
    s1 = jnp.ones((1,), jnp.float32)
    s2 = (1.0 + eps2).reshape(1)

    agg = _sc_segment_sum(x, packed3, zer)
    h = _tc_mlp_bn(s0, x, agg, W0a, b0a, W0b, b0b, gamma, beta)
    agg = _sc_segment_sum(h, packed3, zer)
    h = _tc_mlp_bn(s1, h, agg, W1a, b1a, W1b, b1b, gamma, beta)
    agg = _sc_segment_sum(h, packed3, zer)
    return _tc_mlp(s2, h, agg, W2a, b2a, W2b, b2b)


# speedup vs baseline: 3.5881x; 1.0138x over previous
import functools

import numpy as np
import jax
import jax.numpy as jnp
from jax import lax
from jax.experimental import pallas as pl
from jax.experimental.pallas import tpu as pltpu
from jax.experimental.pallas import tpu_sc as plsc

_N = 10000
_E = 320000
_D = 128

_NC = 2
_NS = 16
_NW = _NC * _NS
_CH = 128
_NCHUNK = 80
_NPAIR = _NCHUNK // 2
_EPW = _NCHUNK * _CH
_E2 = _NW * _EPW
_NPADROW = 128
_NACC = _N + _NPADROW
_RPT = 632
_SHIFT = 14


def _sc_segment_sum(h, packed3, zer):
    mesh = plsc.VectorSubcoreMesh(core_axis_name="c", subcore_axis_name="s")

    def body(h_hbm, packed_hbm, zer_hbm, out_hbm,
             packed_v, src0, dst0, src1, dst1, rows0, rows1, acc_sh,
             sem_g0, sem_g1, sem_s0, sem_s1, sem_i):
        c = lax.axis_index("c")
        s = lax.axis_index("s")
        wid = s * _NC + c

        def on_my_rows(fn, last_n):
            @pl.when(s < _NS - 1)
            def _():
                fn(pl.ds(pl.multiple_of(s * _RPT, 8), _RPT))

            @pl.when(s == _NS - 1)
            def _():
                fn(pl.ds((_NS - 1) * _RPT, last_n))

        on_my_rows(lambda rsl: pltpu.async_copy(zer_hbm.at[rsl], acc_sh.at[rsl],
                                                sem_i),
                   _NACC - 15 * _RPT)
        pltpu.sync_copy(packed_hbm.at[wid], packed_v)
        on_my_rows(lambda rsl: pltpu.make_async_copy(zer_hbm.at[rsl],
                                                     acc_sh.at[rsl], sem_i).wait(),
                   _NACC - 15 * _RPT)
        plsc.subcore_barrier()

        def unpack_src(j, sbuf):
            for k in range(_CH // 16):
                p = packed_v[j, pl.ds(k * 16, 16)]
                sbuf[0, pl.ds(k * 16, 16)] = p & (2**_SHIFT - 1)

        def unpack_dst(j, dbuf):
            for k in range(_CH // 16):
                p = packed_v[j, pl.ds(k * 16, 16)]
                dbuf[0, pl.ds(k * 16, 16)] = lax.shift_right_logical(p, _SHIFT)

        unpack_src(0, src0)
        pltpu.async_copy(h_hbm.at[src0.at[0]], rows0, sem_g0)
        unpack_dst(0, dst0)

        def step(t, carry):
            a = 2 * t
            b = a + 1

            unpack_src(b, src1)

            @pl.when(t > 0)
            def _():
                pltpu.make_async_copy(rows1, acc_sh.at[dst1.at[0]], sem_s1).wait()

            pltpu.async_copy(h_hbm.at[src1.at[0]], rows1, sem_g1)
            unpack_dst(b, dst1)
            pltpu.make_async_copy(h_hbm.at[src0.at[0]], rows0, sem_g0).wait()
            pltpu.async_copy(rows0, acc_sh.at[dst0.at[0]], sem_s0, add=True)

            @pl.when(t < _NPAIR - 1)
            def _():
                unpack_src(a + 2, src0)

            pltpu.make_async_copy(rows0, acc_sh.at[dst0.at[0]], sem_s0).wait()

            @pl.when(t < _NPAIR - 1)
            def _():
                pltpu.async_copy(h_hbm.at[src0.at[0]], rows0, sem_g0)
                unpack_dst(a + 2, dst0)

            pltpu.make_async_copy(h_hbm.at[src1.at[0]], rows1, sem_g1).wait()
            pltpu.async_copy(rows1, acc_sh.at[dst1.at[0]], sem_s1, add=True)
            return carry

        lax.fori_loop(0, _NPAIR, step, 0)
        pltpu.make_async_copy(rows1, acc_sh.at[dst1.at[0]], sem_s1).wait()
        plsc.subcore_barrier()
        on_my_rows(lambda rsl: pltpu.sync_copy(acc_sh.at[rsl], out_hbm.at[c, rsl]),
                   _N - 15 * _RPT)

    f = pl.kernel(
        body,
        out_type=jax.ShapeDtypeStruct((_NC, _N, _D), jnp.float32),
        mesh=mesh,
        scratch_types=[
            pltpu.VMEM((_NCHUNK, _CH), jnp.int32),
            pltpu.VMEM((1, _CH), jnp.int32),
            pltpu.VMEM((1, _CH), jnp.int32),
            pltpu.VMEM((1, _CH), jnp.int32),
            pltpu.VMEM((1, _CH), jnp.int32),
            pltpu.VMEM((_CH, _D), jnp.float32),
            pltpu.VMEM((_CH, _D), jnp.float32),
            pltpu.VMEM_SHARED((_NACC, _D), jnp.float32),
            pltpu.SemaphoreType.DMA,
            pltpu.SemaphoreType.DMA,
            pltpu.SemaphoreType.DMA,
            pltpu.SemaphoreType.DMA,
            pltpu.SemaphoreType.DMA,
        ],
    )
    return f(h, packed3, zer)


_BR = 1000


def _tc_mlp_bn(scale, h, agg2, Wa, ba, Wb, bb, gamma, beta):
    def body(sc_ref, h_ref, a0_ref, a1_ref, wa_ref, ba_ref, wb_ref, bb_ref,
             g_ref, be_ref, o_ref):
        z = sc_ref[0] * h_ref[...] + a0_ref[...] + a1_ref[...]
        z = jnp.dot(z, wa_ref[...], preferred_element_type=jnp.float32)
        z = jnp.maximum(z + ba_ref[...], 0.0)
        y = jnp.dot(z, wb_ref[...], preferred_element_type=jnp.float32)
        y = y + bb_ref[...]
        y = y * (g_ref[...] * np.float32(1.0 / np.sqrt(1.0 + 1e-5))) + be_ref[...]
        o_ref[...] = jnp.where(y >= 0.0, y, 0.01 * y)

    dout = Wb.shape[1]
    return pl.pallas_call(
        body,
        grid=(_N // _BR,),
        in_specs=[
            pl.BlockSpec(memory_space=pltpu.SMEM),
            pl.BlockSpec((_BR, _D), lambda i: (i, 0)),
            pl.BlockSpec((_BR, _D), lambda i: (i, 0)),
            pl.BlockSpec((_BR, _D), lambda i: (i, 0)),
            pl.BlockSpec((_D, _D), lambda i: (0, 0)),
            pl.BlockSpec((1, _D), lambda i: (0, 0)),
            pl.BlockSpec((_D, dout), lambda i: (0, 0)),
            pl.BlockSpec((1, dout), lambda i: (0, 0)),
            pl.BlockSpec((1, dout), lambda i: (0, 0)),
            pl.BlockSpec((1, dout), lambda i: (0, 0)),
        ],
        out_specs=pl.BlockSpec((_BR, dout), lambda i: (i, 0)),
        out_shape=jax.ShapeDtypeStruct((_N, dout), jnp.float32),
    )(scale, h, agg2[0], agg2[1], Wa, ba.reshape(1, -1), Wb,
      bb.reshape(1, -1), gamma.reshape(1, -1), beta.reshape(1, -1))


def _tc_mlp(scale, h, agg2, Wa, ba, Wb, bb):
    def body(sc_ref, h_ref, a0_ref, a1_ref, wa_ref, ba_ref, wb_ref, bb_ref,
             o_ref):
        z = sc_ref[0] * h_ref[...] + a0_ref[...] + a1_ref[...]
        z = jnp.dot(z, wa_ref[...], preferred_element_type=jnp.float32)
        z = jnp.maximum(z + ba_ref[...], 0.0)
        y = jnp.dot(z, wb_ref[...], preferred_element_type=jnp.float32)
        o_ref[...] = y + bb_ref[...]

    dout = Wb.shape[1]
    return pl.pallas_call(
        body,
        grid=(_N // _BR,),
        in_specs=[
            pl.BlockSpec(memory_space=pltpu.SMEM),
            pl.BlockSpec((_BR, _D), lambda i: (i, 0)),
            pl.BlockSpec((_BR, _D), lambda i: (i, 0)),
            pl.BlockSpec((_BR, _D), lambda i: (i, 0)),
            pl.BlockSpec((_D, _D), lambda i: (0, 0)),
            pl.BlockSpec((1, _D), lambda i: (0, 0)),
            pl.BlockSpec((_D, dout), lambda i: (0, 0)),
            pl.BlockSpec((1, dout), lambda i: (0, 0)),
        ],
        out_specs=pl.BlockSpec((_BR, dout), lambda i: (i, 0)),
        out_shape=jax.ShapeDtypeStruct((_N, dout), jnp.float32),
    )(scale, h, agg2[0], agg2[1], Wa, ba.reshape(1, -1), Wb,
      bb.reshape(1, -1))


def kernel(x, edge_index, W0a, b0a, W0b, b0b, eps0, W1a, b1a, W1b, b1b,
           W2a, b2a, W2b, b2b, eps2, gamma, beta):
    packed = edge_index[0] + edge_index[1] * (2**_SHIFT)
    _pi = np.arange(_E2 - _E)
    pad = jnp.asarray(_pi * 13 % _N + (_N + _pi % _NPADROW) * (2**_SHIFT),
                      jnp.int32)
    packed3 = jnp.concatenate([packed, pad]).reshape(_NW, _NCHUNK, _CH)
    zer = jnp.zeros((_NACC, _D), jnp.float32)

    s0 = (1.0 + eps0).reshape(1)
    s1 = jnp.ones((1,), jnp.float32)
    s2 = (1.0 + eps2).reshape(1)

    agg = _sc_segment_sum(x, packed3, zer)
    h = _tc_mlp_bn(s0, x, agg, W0a, b0a, W0b, b0b, gamma, beta)
    agg = _sc_segment_sum(h, packed3, zer)
    h = _tc_mlp_bn(s1, h, agg, W1a, b1a, W1b, b1b, gamma, beta)
    agg = _sc_segment_sum(h, packed3, zer)
    return _tc_mlp(s2, h, agg, W2a, b2a, W2b, b2b)


# speedup vs baseline: 3.6626x; 1.0208x over previous
import numpy as np
import jax
import jax.numpy as jnp
from jax import lax
from jax.experimental import pallas as pl
from jax.experimental.pallas import tpu as pltpu
from jax.experimental.pallas import tpu_sc as plsc

_N = 10000
_E = 320000
_D = 128

_NC = 2
_NS = 16
_NW = _NC * _NS
_CH = 128
_NCHUNK = 80
_NPAIR = _NCHUNK // 2
_EPW = _NCHUNK * _CH
_E2 = _NW * _EPW
_NPADROW = 128
_NACC = _N + _NPADROW
_RPT = 632
_SHIFT = 14


def _sc_segment_sum(h, packed3, zer):
    mesh = plsc.VectorSubcoreMesh(core_axis_name="c", subcore_axis_name="s")

    def body(h_hbm, packed_hbm, zer_hbm, out_hbm,
             packed_v, src0, dst0, src1, dst1, rows0, rows1, acc_sh,
             sem_g0, sem_g1, sem_s0, sem_s1, sem_i):
        c = lax.axis_index("c")
        s = lax.axis_index("s")
        wid = s * _NC + c

        def on_my_rows(fn, last_n):
            @pl.when(s < _NS - 1)
            def _():
                fn(pl.ds(pl.multiple_of(s * _RPT, 8), _RPT))

            @pl.when(s == _NS - 1)
            def _():
                fn(pl.ds((_NS - 1) * _RPT, last_n))

        on_my_rows(lambda rsl: pltpu.async_copy(zer_hbm.at[rsl], acc_sh.at[rsl],
                                                sem_i),
                   _NACC - 15 * _RPT)
        pltpu.sync_copy(packed_hbm.at[wid], packed_v)
        on_my_rows(lambda rsl: pltpu.make_async_copy(zer_hbm.at[rsl],
                                                     acc_sh.at[rsl], sem_i).wait(),
                   _NACC - 15 * _RPT)
        plsc.subcore_barrier()

        def unpack_src(j, sbuf):
            for k in range(_CH // 16):
                p = packed_v[j, pl.ds(k * 16, 16)]
                sbuf[0, pl.ds(k * 16, 16)] = p & (2**_SHIFT - 1)

        def unpack_dst(j, dbuf):
            for k in range(_CH // 16):
                p = packed_v[j, pl.ds(k * 16, 16)]
                dbuf[0, pl.ds(k * 16, 16)] = lax.shift_right_logical(p, _SHIFT)

        unpack_src(0, src0)
        pltpu.async_copy(h_hbm.at[src0.at[0]], rows0, sem_g0)
        unpack_dst(0, dst0)

        def step(t, carry):
            a = 2 * t
            b = a + 1

            unpack_src(b, src1)

            @pl.when(t > 0)
            def _():
                pltpu.make_async_copy(rows1, acc_sh.at[dst1.at[0]], sem_s1).wait()

            pltpu.async_copy(h_hbm.at[src1.at[0]], rows1, sem_g1)
            unpack_dst(b, dst1)
            pltpu.make_async_copy(h_hbm.at[src0.at[0]], rows0, sem_g0).wait()
            pltpu.async_copy(rows0, acc_sh.at[dst0.at[0]], sem_s0, add=True)

            @pl.when(t < _NPAIR - 1)
            def _():
                unpack_src(a + 2, src0)

            pltpu.make_async_copy(rows0, acc_sh.at[dst0.at[0]], sem_s0).wait()

            @pl.when(t < _NPAIR - 1)
            def _():
                pltpu.async_copy(h_hbm.at[src0.at[0]], rows0, sem_g0)
                unpack_dst(a + 2, dst0)

            pltpu.make_async_copy(h_hbm.at[src1.at[0]], rows1, sem_g1).wait()
            pltpu.async_copy(rows1, acc_sh.at[dst1.at[0]], sem_s1, add=True)
            return carry

        lax.fori_loop(0, _NPAIR, step, 0)
        pltpu.make_async_copy(rows1, acc_sh.at[dst1.at[0]], sem_s1).wait()
        plsc.subcore_barrier()
        on_my_rows(lambda rsl: pltpu.sync_copy(acc_sh.at[rsl], out_hbm.at[c, rsl]),
                   _N - 15 * _RPT)

    f = pl.kernel(
        body,
        out_type=jax.ShapeDtypeStruct((_NC, _N, _D), jnp.float32),
        mesh=mesh,
        scratch_types=[
            pltpu.VMEM((_NCHUNK, _CH), jnp.int32),
            pltpu.VMEM((1, _CH), jnp.int32),
            pltpu.VMEM((1, _CH), jnp.int32),
            pltpu.VMEM((1, _CH), jnp.int32),
            pltpu.VMEM((1, _CH), jnp.int32),
            pltpu.VMEM((_CH, _D), jnp.float32),
            pltpu.VMEM((_CH, _D), jnp.float32),
            pltpu.VMEM_SHARED((_NACC, _D), jnp.float32),
            pltpu.SemaphoreType.DMA,
            pltpu.SemaphoreType.DMA,
            pltpu.SemaphoreType.DMA,
            pltpu.SemaphoreType.DMA,
            pltpu.SemaphoreType.DMA,
        ],
    )
    return f(h, packed3, zer)


_BR = 2000


def _tc_mlp_bn(scale, h, agg2, Wa, ba, Wb, bb, gamma, beta):
    def body(sc_ref, h_ref, a0_ref, a1_ref, wa_ref, ba_ref, wb_ref, bb_ref,
             g_ref, be_ref, o_ref):
        z = sc_ref[0] * h_ref[...] + a0_ref[...] + a1_ref[...]
        z = jnp.dot(z, wa_ref[...], preferred_element_type=jnp.float32)
        z = jnp.maximum(z + ba_ref[...], 0.0)
        y = jnp.dot(z, wb_ref[...], preferred_element_type=jnp.float32)
        y = y + bb_ref[...]
        y = y * (g_ref[...] * np.float32(1.0 / np.sqrt(1.0 + 1e-5))) + be_ref[...]
        o_ref[...] = jnp.where(y >= 0.0, y, 0.01 * y)

    dout = Wb.shape[1]
    return pl.pallas_call(
        body,
        grid=(_N // _BR,),
        in_specs=[
            pl.BlockSpec(memory_space=pltpu.SMEM),
            pl.BlockSpec((_BR, _D), lambda i: (i, 0)),
            pl.BlockSpec((_BR, _D), lambda i: (i, 0)),
            pl.BlockSpec((_BR, _D), lambda i: (i, 0)),
            pl.BlockSpec((_D, _D), lambda i: (0, 0)),
            pl.BlockSpec((1, _D), lambda i: (0, 0)),
            pl.BlockSpec((_D, dout), lambda i: (0, 0)),
            pl.BlockSpec((1, dout), lambda i: (0, 0)),
            pl.BlockSpec((1, dout), lambda i: (0, 0)),
            pl.BlockSpec((1, dout), lambda i: (0, 0)),
        ],
        out_specs=pl.BlockSpec((_BR, dout), lambda i: (i, 0)),
        out_shape=jax.ShapeDtypeStruct((_N, dout), jnp.float32),
    )(scale, h, agg2[0], agg2[1], Wa, ba.reshape(1, -1), Wb,
      bb.reshape(1, -1), gamma.reshape(1, -1), beta.reshape(1, -1))


def _tc_mlp(scale, h, agg2, Wa, ba, Wb, bb):
    def body(sc_ref, h_ref, a0_ref, a1_ref, wa_ref, ba_ref, wb_ref, bb_ref,
             o_ref):
        z = sc_ref[0] * h_ref[...] + a0_ref[...] + a1_ref[...]
        z = jnp.dot(z, wa_ref[...], preferred_element_type=jnp.float32)
        z = jnp.maximum(z + ba_ref[...], 0.0)
        y = jnp.dot(z, wb_ref[...], preferred_element_type=jnp.float32)
        o_ref[...] = y + bb_ref[...]

    dout = Wb.shape[1]
    return pl.pallas_call(
        body,
        grid=(_N // _BR,),
        in_specs=[
            pl.BlockSpec(memory_space=pltpu.SMEM),
            pl.BlockSpec((_BR, _D), lambda i: (i, 0)),
            pl.BlockSpec((_BR, _D), lambda i: (i, 0)),
            pl.BlockSpec((_BR, _D), lambda i: (i, 0)),
            pl.BlockSpec((_D, _D), lambda i: (0, 0)),
            pl.BlockSpec((1, _D), lambda i: (0, 0)),
            pl.BlockSpec((_D, dout), lambda i: (0, 0)),
            pl.BlockSpec((1, dout), lambda i: (0, 0)),
        ],
        out_specs=pl.BlockSpec((_BR, dout), lambda i: (i, 0)),
        out_shape=jax.ShapeDtypeStruct((_N, dout), jnp.float32),
    )(scale, h, agg2[0], agg2[1], Wa, ba.reshape(1, -1), Wb,
      bb.reshape(1, -1))


def kernel(x, edge_index, W0a, b0a, W0b, b0b, eps0, W1a, b1a, W1b, b1b,
           W2a, b2a, W2b, b2b, eps2, gamma, beta):
    packed = edge_index[0] + edge_index[1] * (2**_SHIFT)
    _pi = np.arange(_E2 - _E)
    pad = jnp.asarray(_pi * 13 % _N + (_N + _pi % _NPADROW) * (2**_SHIFT),
                      jnp.int32)
    packed3 = jnp.concatenate([packed, pad]).reshape(_NW, _NCHUNK, _CH)
    zer = jnp.zeros((_NACC, _D), jnp.float32)

    s0 = (1.0 + eps0).reshape(1)
    s1 = jnp.ones((1,), jnp.float32)
    s2 = (1.0 + eps2).reshape(1)

    agg = _sc_segment_sum(x, packed3, zer)
    h = _tc_mlp_bn(s0, x, agg, W0a, b0a, W0b, b0b, gamma, beta)
    agg = _sc_segment_sum(h, packed3, zer)
    h = _tc_mlp_bn(s1, h, agg, W1a, b1a, W1b, b1b, gamma, beta)
    agg = _sc_segment_sum(h, packed3, zer)
    return _tc_mlp(s2, h, agg, W2a, b2a, W2b, b2b)
